# baseline (speedup 1.0000x reference)
import functools

import jax
import jax.numpy as jnp
from jax import lax
from jax.experimental import pallas as pl
from jax.experimental.pallas import tpu as pltpu
from jax.experimental.pallas import tpu_sc as plsc

D = 32
B = 16384
NC, NS = 2, 16
NW = NC * NS
B_PER_W = B // NW
GRP = 16
N_GRP = B_PER_W // GRP


def _gather_body(table_hbm, idx_hbm, out_hbm, idx_v, rows_v, sem):
    wid = lax.axis_index("s") * NC + lax.axis_index("c")
    base = wid * B_PER_W
    pltpu.sync_copy(idx_hbm.at[pl.ds(base, B_PER_W)], idx_v)
    lane = jnp.arange(GRP, dtype=jnp.int32)

    for g in range(N_GRP):
        vec = idx_v[g * GRP : (g + 1) * GRP]
        for k in range(GRP):
            row = jnp.max(jnp.where(lane == k, vec, 0), axis=0)
            pltpu.async_copy(
                table_hbm.at[pl.ds(row, 1)],
                rows_v.at[pl.ds(g * GRP + k, 1)],
                sem,
            )
        if g > 0:
            pltpu.make_async_copy(
                table_hbm.at[pl.ds(0, GRP)],
                rows_v.at[pl.ds((g - 1) * GRP, GRP)],
                sem,
            ).wait()
    pltpu.make_async_copy(
        table_hbm.at[pl.ds(0, GRP)],
        rows_v.at[pl.ds((N_GRP - 1) * GRP, GRP)],
        sem,
    ).wait()
    pltpu.sync_copy(rows_v, out_hbm.at[pl.ds(base, B_PER_W)])


@jax.jit
def kernel(process_indices, table):
    idx = process_indices.astype(jnp.int32)
    mesh = plsc.VectorSubcoreMesh(core_axis_name="c", subcore_axis_name="s")
    k = functools.partial(
        pl.kernel,
        mesh=mesh,
        out_type=jax.ShapeDtypeStruct((B, D), jnp.float32),
        scratch_types=[
            pltpu.VMEM((B_PER_W,), jnp.int32),
            pltpu.VMEM((B_PER_W, D), jnp.float32),
            pltpu.SemaphoreType.DMA,
        ],
        compiler_params=pltpu.CompilerParams(
            use_tc_tiling_on_sc=True,
            needs_layout_passes=False,
            skip_device_barrier=True,
            disable_bounds_checks=True,
            disable_semaphore_checks=True,
        ),
    )(_gather_body)
    return k(table, idx)


# speedup vs baseline: 1.1482x; 1.1482x over previous
import functools

import jax
import jax.numpy as jnp
from jax import lax
from jax.experimental import pallas as pl
from jax.experimental.pallas import tpu as pltpu
from jax.experimental.pallas import tpu_sc as plsc

D = 32
B = 16384
NC, NS = 2, 16
NW = NC * NS
B_PER_W = B // NW
GRP = 32
N_GRP = B_PER_W // GRP


def _gather_body(table_hbm, idx_hbm, out_hbm, idx_v, rows_v, sem):
    wid = lax.axis_index("s") * NC + lax.axis_index("c")
    base = wid * B_PER_W
    pltpu.sync_copy(idx_hbm.at[pl.ds(base, B_PER_W)], idx_v)
    lane = jnp.arange(16, dtype=jnp.int32)

    def group(g, carry):
        for h in range(GRP // 16):
            vec = idx_v[pl.ds(g * GRP + h * 16, 16)]
            for k in range(16):
                row = jnp.max(jnp.where(lane == k, vec, 0), axis=0)
                pltpu.async_copy(
                    table_hbm.at[pl.ds(row, 1)],
                    rows_v.at[pl.ds(g * GRP + h * 16 + k, 1)],
                    sem,
                )
        @pl.when(g > 0)
        def _():
            pltpu.make_async_copy(
                table_hbm.at[pl.ds(0, GRP)],
                rows_v.at[pl.ds((g - 1) * GRP, GRP)],
                sem,
            ).wait()

        return carry

    lax.fori_loop(0, N_GRP, group, 0)
    pltpu.make_async_copy(
        table_hbm.at[pl.ds(0, GRP)],
        rows_v.at[pl.ds((N_GRP - 1) * GRP, GRP)],
        sem,
    ).wait()
    pltpu.sync_copy(rows_v, out_hbm.at[pl.ds(base, B_PER_W)])


@jax.jit
def kernel(process_indices, table):
    idx = process_indices.astype(jnp.int32)
    mesh = plsc.VectorSubcoreMesh(core_axis_name="c", subcore_axis_name="s")
    k = functools.partial(
        pl.kernel,
        mesh=mesh,
        out_type=jax.ShapeDtypeStruct((B, D), jnp.float32),
        scratch_types=[
            pltpu.VMEM((B_PER_W,), jnp.int32),
            pltpu.VMEM((B_PER_W, D), jnp.float32),
            pltpu.SemaphoreType.DMA,
        ],
        compiler_params=pltpu.CompilerParams(
            use_tc_tiling_on_sc=True,
            needs_layout_passes=False,
            skip_device_barrier=True,
            disable_bounds_checks=True,
            disable_semaphore_checks=True,
        ),
    )(_gather_body)
    return k(table, idx)


# speedup vs baseline: 1.2006x; 1.0457x over previous
import functools

import jax
import jax.numpy as jnp
from jax import lax
from jax.experimental import pallas as pl
from jax.experimental.pallas import tpu as pltpu
from jax.experimental.pallas import tpu_sc as plsc

D = 32
B = 16384
NC, NS = 2, 16
NW = NC * NS
B_PER_W = B // NW
GRP = 64
N_GRP = B_PER_W // GRP


def _gather_body(table_hbm, idx_hbm, out_hbm, idx_v, rows_v, sem):
    wid = lax.axis_index("s") * NC + lax.axis_index("c")
    base = wid * B_PER_W
    pltpu.sync_copy(idx_hbm.at[pl.ds(base, B_PER_W)], idx_v)
    lane = jnp.arange(16, dtype=jnp.int32)

    def group(g, carry):
        for h in range(GRP // 16):
            vec = idx_v[pl.ds(g * GRP + h * 16, 16)]
            for k in range(16):
                row = jnp.max(jnp.where(lane == k, vec, 0), axis=0)
                pltpu.async_copy(
                    table_hbm.at[pl.ds(row, 1)],
                    rows_v.at[pl.ds(g * GRP + h * 16 + k, 1)],
                    sem,
                )
        @pl.when(g > 0)
        def _():
            pltpu.make_async_copy(
                table_hbm.at[pl.ds(0, GRP)],
                rows_v.at[pl.ds((g - 1) * GRP, GRP)],
                sem,
            ).wait()

        return carry

    lax.fori_loop(0, N_GRP, group, 0)
    pltpu.make_async_copy(
        table_hbm.at[pl.ds(0, GRP)],
        rows_v.at[pl.ds((N_GRP - 1) * GRP, GRP)],
        sem,
    ).wait()
    pltpu.sync_copy(rows_v, out_hbm.at[pl.ds(base, B_PER_W)])


@jax.jit
def kernel(process_indices, table):
    idx = process_indices.astype(jnp.int32)
    mesh = plsc.VectorSubcoreMesh(core_axis_name="c", subcore_axis_name="s")
    k = functools.partial(
        pl.kernel,
        mesh=mesh,
        out_type=jax.ShapeDtypeStruct((B, D), jnp.float32),
        scratch_types=[
            pltpu.VMEM((B_PER_W,), jnp.int32),
            pltpu.VMEM((B_PER_W, D), jnp.float32),
            pltpu.SemaphoreType.DMA,
        ],
        compiler_params=pltpu.CompilerParams(
            use_tc_tiling_on_sc=True,
            needs_layout_passes=False,
            skip_device_barrier=True,
            disable_bounds_checks=True,
            disable_semaphore_checks=True,
        ),
    )(_gather_body)
    return k(table, idx)


# speedup vs baseline: 1.2056x; 1.0042x over previous
import functools

import jax
import jax.numpy as jnp
from jax import lax
from jax.experimental import pallas as pl
from jax.experimental.pallas import tpu as pltpu
from jax.experimental.pallas import tpu_sc as plsc

D = 32
B = 16384
NC, NS = 2, 16
NW = NC * NS
B_PER_W = B // NW
GRP = 128
N_GRP = B_PER_W // GRP


def _gather_body(table_hbm, idx_hbm, out_hbm, idx_v, rows_v, sem):
    wid = lax.axis_index("s") * NC + lax.axis_index("c")
    base = wid * B_PER_W
    pltpu.sync_copy(idx_hbm.at[pl.ds(base, B_PER_W)], idx_v)
    lane = jnp.arange(16, dtype=jnp.int32)

    def group(g, carry):
        for h in range(GRP // 16):
            vec = idx_v[pl.ds(g * GRP + h * 16, 16)]
            for k in range(16):
                row = jnp.max(jnp.where(lane == k, vec, 0), axis=0)
                pltpu.async_copy(
                    table_hbm.at[pl.ds(row, 1)],
                    rows_v.at[pl.ds(g * GRP + h * 16 + k, 1)],
                    sem,
                )
        @pl.when(g > 0)
        def _():
            pltpu.make_async_copy(
                table_hbm.at[pl.ds(0, GRP)],
                rows_v.at[pl.ds((g - 1) * GRP, GRP)],
                sem,
            ).wait()

        return carry

    lax.fori_loop(0, N_GRP, group, 0)
    pltpu.make_async_copy(
        table_hbm.at[pl.ds(0, GRP)],
        rows_v.at[pl.ds((N_GRP - 1) * GRP, GRP)],
        sem,
    ).wait()
    pltpu.sync_copy(rows_v, out_hbm.at[pl.ds(base, B_PER_W)])


@jax.jit
def kernel(process_indices, table):
    idx = process_indices.astype(jnp.int32)
    mesh = plsc.VectorSubcoreMesh(core_axis_name="c", subcore_axis_name="s")
    k = functools.partial(
        pl.kernel,
        mesh=mesh,
        out_type=jax.ShapeDtypeStruct((B, D), jnp.float32),
        scratch_types=[
            pltpu.VMEM((B_PER_W,), jnp.int32),
            pltpu.VMEM((B_PER_W, D), jnp.float32),
            pltpu.SemaphoreType.DMA,
        ],
        compiler_params=pltpu.CompilerParams(
            use_tc_tiling_on_sc=True,
            needs_layout_passes=False,
            skip_device_barrier=True,
            disable_bounds_checks=True,
            disable_semaphore_checks=True,
        ),
    )(_gather_body)
    return k(table, idx)


# speedup vs baseline: 1.2129x; 1.0060x over previous
import functools

import jax
import jax.numpy as jnp
from jax import lax
from jax.experimental import pallas as pl
from jax.experimental.pallas import tpu as pltpu
from jax.experimental.pallas import tpu_sc as plsc

D = 32
B = 16384
NC, NS = 2, 16
NW = NC * NS
B_PER_W = B // NW
GRP = 128
N_GRP = B_PER_W // GRP


def _gather_body(table_hbm, idx_hbm, out_hbm, idx_v, rows_v, sem):
    wid = lax.axis_index("s") * NC + lax.axis_index("c")
    base = wid * B_PER_W
    pltpu.sync_copy(idx_hbm.at[pl.ds(base, B_PER_W)], idx_v)
    lane = jnp.arange(16, dtype=jnp.int32)

    def group(g, carry):
        for h in range(GRP // 16):
            vec = idx_v[pl.ds(g * GRP + h * 16, 16)]
            for k in range(16):
                row = jnp.max(jnp.where(lane == k, vec, 0), axis=0)
                pltpu.async_copy(
                    table_hbm.at[pl.ds(row, 1)],
                    rows_v.at[pl.ds(g * GRP + h * 16 + k, 1)],
                    sem,
                )
        return carry

    lax.fori_loop(0, N_GRP, group, 0)
    pltpu.make_async_copy(
        table_hbm.at[pl.ds(0, B_PER_W)],
        rows_v,
        sem,
    ).wait()
    pltpu.sync_copy(rows_v, out_hbm.at[pl.ds(base, B_PER_W)])


@jax.jit
def kernel(process_indices, table):
    idx = process_indices.astype(jnp.int32)
    mesh = plsc.VectorSubcoreMesh(core_axis_name="c", subcore_axis_name="s")
    k = functools.partial(
        pl.kernel,
        mesh=mesh,
        out_type=jax.ShapeDtypeStruct((B, D), jnp.float32),
        scratch_types=[
            pltpu.VMEM((B_PER_W,), jnp.int32),
            pltpu.VMEM((B_PER_W, D), jnp.float32),
            pltpu.SemaphoreType.DMA,
        ],
        compiler_params=pltpu.CompilerParams(
            use_tc_tiling_on_sc=True,
            needs_layout_passes=False,
            skip_device_barrier=True,
            disable_bounds_checks=True,
            disable_semaphore_checks=True,
        ),
    )(_gather_body)
    return k(table, idx)


# speedup vs baseline: 1.2258x; 1.0107x over previous
import functools

import jax
import jax.numpy as jnp
from jax import lax
from jax.experimental import pallas as pl
from jax.experimental.pallas import tpu as pltpu
from jax.experimental.pallas import tpu_sc as plsc

D = 32
B = 16384
NC, NS = 2, 16
NW = NC * NS
B_PER_W = B // NW
GRP = 32
N_GRP = B_PER_W // GRP


def _gather_body(table_hbm, idx_hbm, out_hbm, idx_v, rows_v, sem):
    wid = lax.axis_index("s") * NC + lax.axis_index("c")
    base = wid * B_PER_W
    pltpu.sync_copy(idx_hbm.at[pl.ds(base, B_PER_W)], idx_v)
    lane = jnp.arange(16, dtype=jnp.int32)

    def group(g, carry):
        for h in range(GRP // 16):
            vec = idx_v[pl.ds(g * GRP + h * 16, 16)]
            for k in range(16):
                row = jnp.max(jnp.where(lane == k, vec, 0), axis=0)
                pltpu.async_copy(
                    table_hbm.at[pl.ds(row, 1)],
                    rows_v.at[pl.ds(g * GRP + h * 16 + k, 1)],
                    sem,
                )
        return carry

    lax.fori_loop(0, N_GRP, group, 0)
    pltpu.make_async_copy(
        table_hbm.at[pl.ds(0, B_PER_W)],
        rows_v,
        sem,
    ).wait()
    pltpu.sync_copy(rows_v, out_hbm.at[pl.ds(base, B_PER_W)])


@jax.jit
def kernel(process_indices, table):
    idx = process_indices.astype(jnp.int32)
    mesh = plsc.VectorSubcoreMesh(core_axis_name="c", subcore_axis_name="s")
    k = functools.partial(
        pl.kernel,
        mesh=mesh,
        out_type=jax.ShapeDtypeStruct((B, D), jnp.float32),
        scratch_types=[
            pltpu.VMEM((B_PER_W,), jnp.int32),
            pltpu.VMEM((B_PER_W, D), jnp.float32),
            pltpu.SemaphoreType.DMA,
        ],
        compiler_params=pltpu.CompilerParams(
            use_tc_tiling_on_sc=True,
            needs_layout_passes=False,
            skip_device_barrier=True,
            disable_bounds_checks=True,
            disable_semaphore_checks=True,
        ),
    )(_gather_body)
    return k(table, idx)
